# baseline (speedup 1.0000x reference)
import jax
import jax.numpy as jnp
from jax.experimental import pallas as pl

B, N, H = 16384, 1024, 256
TILE_B = 1024
LOG2E = 1.4426950408889634


def _retrieval_kernel(z_ref, mem_ref, out_ref):
    z = z_ref[...]
    mem = mem_ref[...]

    z32 = z.astype(jnp.float32)
    z_norm = (z32 * jax.lax.rsqrt(jnp.maximum(jnp.sum(z32 * z32, axis=1, keepdims=True), 1e-24))).astype(jnp.bfloat16)

    m_inv = jax.lax.rsqrt(jnp.maximum(jnp.sum(mem * mem, axis=1, keepdims=True), 1e-24))
    key = (mem * (m_inv * LOG2E)).astype(jnp.bfloat16)

    sim = jax.lax.dot_general(
        z_norm, key,
        (((1,), (1,)), ((), ())),
        preferred_element_type=jnp.float32,
    )

    e = jnp.exp2(sim.astype(jnp.bfloat16))
    inv_sum = 1.0 / jnp.sum(e, axis=1, keepdims=True, dtype=jnp.float32)

    acc = jnp.dot(e, mem.astype(jnp.bfloat16), preferred_element_type=jnp.float32)
    out_ref[...] = acc * inv_sum


def kernel(z, memory):
    z_bf = z.astype(jnp.bfloat16)
    return pl.pallas_call(
        _retrieval_kernel,
        grid=(B // TILE_B,),
        in_specs=[
            pl.BlockSpec((TILE_B, H), lambda i: (i, 0)),
            pl.BlockSpec((N, H), lambda i: (0, 0)),
        ],
        out_specs=pl.BlockSpec((TILE_B, H), lambda i: (i, 0)),
        out_shape=jax.ShapeDtypeStruct((B, H), jnp.float32),
    )(z_bf, memory)


# speedup vs baseline: 1.4704x; 1.4704x over previous
import jax
import jax.numpy as jnp
from jax.experimental import pallas as pl

B, N, H = 16384, 1024, 256
TILE_B = 2048
LOG2E = 1.4426950408889634


def _retrieval_kernel(z_ref, mem_ref, out_ref):
    z = z_ref[...]
    mem = mem_ref[...]

    z_norm = z * jax.lax.rsqrt(jnp.maximum(jnp.sum(z * z, axis=1, keepdims=True), 1e-24))

    m_inv = jax.lax.rsqrt(jnp.maximum(jnp.sum(mem * mem, axis=1, keepdims=True), 1e-24))
    key = (mem * (m_inv * LOG2E)).astype(jnp.bfloat16)

    sim = jax.lax.dot_general(
        z_norm.astype(jnp.bfloat16), key,
        (((1,), (1,)), ((), ())),
        preferred_element_type=jnp.float32,
    )

    e = jnp.exp2(sim.astype(jnp.bfloat16))
    inv_sum = 1.0 / jnp.sum(e, axis=1, keepdims=True, dtype=jnp.float32)

    acc = jnp.dot(e, mem.astype(jnp.bfloat16), preferred_element_type=jnp.float32)
    out_ref[...] = acc * inv_sum


def kernel(z, memory):
    return pl.pallas_call(
        _retrieval_kernel,
        grid=(B // TILE_B,),
        in_specs=[
            pl.BlockSpec((TILE_B, H), lambda i: (i, 0)),
            pl.BlockSpec((N, H), lambda i: (0, 0)),
        ],
        out_specs=pl.BlockSpec((TILE_B, H), lambda i: (i, 0)),
        out_shape=jax.ShapeDtypeStruct((B, H), jnp.float32),
    )(z, memory)


# speedup vs baseline: 1.5065x; 1.0246x over previous
import jax
import jax.numpy as jnp
from jax.experimental import pallas as pl

B, N, H = 16384, 1024, 256
TILE_B = 4096
LOG2E = 1.4426950408889634


def _retrieval_kernel(z_ref, mem_ref, out_ref):
    z = z_ref[...]
    mem = mem_ref[...]

    z_norm = z * jax.lax.rsqrt(jnp.maximum(jnp.sum(z * z, axis=1, keepdims=True), 1e-24))

    m_inv = jax.lax.rsqrt(jnp.maximum(jnp.sum(mem * mem, axis=1, keepdims=True), 1e-24))
    key = (mem * (m_inv * LOG2E)).astype(jnp.bfloat16)

    sim = jax.lax.dot_general(
        z_norm.astype(jnp.bfloat16), key,
        (((1,), (1,)), ((), ())),
        preferred_element_type=jnp.float32,
    )

    e = jnp.exp2(sim.astype(jnp.bfloat16))
    inv_sum = 1.0 / jnp.sum(e, axis=1, keepdims=True, dtype=jnp.float32)

    acc = jnp.dot(e, mem.astype(jnp.bfloat16), preferred_element_type=jnp.float32)
    out_ref[...] = acc * inv_sum


def kernel(z, memory):
    return pl.pallas_call(
        _retrieval_kernel,
        grid=(B // TILE_B,),
        in_specs=[
            pl.BlockSpec((TILE_B, H), lambda i: (i, 0)),
            pl.BlockSpec((N, H), lambda i: (0, 0)),
        ],
        out_specs=pl.BlockSpec((TILE_B, H), lambda i: (i, 0)),
        out_shape=jax.ShapeDtypeStruct((B, H), jnp.float32),
    )(z, memory)
